# Initial kernel scaffold; baseline (speedup 1.0000x reference)
import jax
import jax.numpy as jnp
from jax.experimental import pallas as pl


def kernel(input_tensor, conv_match_w, bn1_g, bn1_b, conv_asm_w, bn2_g, bn2_b, bn3_g, bn3_b, random_rotations):
    raise NotImplementedError("write your pallas kernel here")


# speedup vs baseline: 4.8092x; 4.8092x over previous
import functools

import jax
import jax.numpy as jnp
from jax import lax
from jax.experimental import pallas as pl
from jax.experimental.pallas import tpu as pltpu
from jax.experimental.pallas import tpu_sc as plsc

N = 4
C = 64
L = 8192
CR = 16
H = 4
CHUNK = 8
J = H * L
NJ = N * J
NBP = 136
TB = 2048
NBL = J // TB
KT = 32
RT = KT * CHUNK
NT = (L // CHUNK) // KT
NH = N * H
F32 = jnp.float32
NEG = -1e30


def _f32(x):
    return x.astype(F32)


TLE = 2048
NTL = L // TLE
_LC8 = L // 8


def _shifted(xc, xprev8, xnext8, t):
    row = lax.broadcasted_iota(jnp.int32, (TLE, 1), 0)
    prev_row = xprev8[0, 0, 7:8, :] * jnp.where(t == 0, 0.0, 1.0)
    next_row = xnext8[0, 0, 0:1, :] * jnp.where(t == NTL - 1, 0.0, 1.0)
    xm1 = jnp.where(row == 0, prev_row, pltpu.roll(xc, 1, 0))
    xp1 = jnp.where(row == TLE - 1, next_row, pltpu.roll(xc, TLE - 1, 0))
    return xm1, xp1


def _conv_outs(xc, xprev8, xnext8, t, w1, w2):
    xm1, xp1 = _shifted(xc, xprev8, xnext8, t)
    xcat = jnp.concatenate([xm1, xc, xp1], axis=1)
    o1 = jnp.dot(xcat, w1[...], preferred_element_type=F32)
    o2 = jnp.dot(xc, w2[...], preferred_element_type=F32)
    return o1, o2


def _stats_body(xin, xp8, xn8, w1, w2, stats):
    t = pl.program_id(1)
    o1, o2 = _conv_outs(xin[0], xp8, xn8, t, w1, w2)
    s1 = jnp.sum(o1, axis=0, keepdims=True)
    q1 = jnp.sum(o1 * o1, axis=0, keepdims=True)
    s2 = jnp.sum(o2, axis=0, keepdims=True)
    q2 = jnp.sum(o2 * o2, axis=0, keepdims=True)
    z1 = jnp.zeros((1, 128 - CR), F32)
    z2 = jnp.zeros((1, 128 - C), F32)
    blk = jnp.concatenate([
        jnp.concatenate([s1, z1], axis=1),
        jnp.concatenate([q1, z1], axis=1),
        jnp.concatenate([s2, z2], axis=1),
        jnp.concatenate([q2, z2], axis=1),
        jnp.zeros((4, 128), F32),
    ], axis=0)
    first = jnp.logical_and(pl.program_id(0) == 0, t == 0)

    @pl.when(first)
    def _():
        stats[...] = blk

    @pl.when(jnp.logical_not(first))
    def _():
        stats[...] = stats[...] + blk


def _halo_specs():
    tlc = TLE // 8
    return [
        pl.BlockSpec((1, TLE, C), lambda n, t: (n, t, 0)),
        pl.BlockSpec((1, 1, 8, C),
                     lambda n, t: (n, (t * tlc - 1) % _LC8, 0, 0)),
        pl.BlockSpec((1, 1, 8, C),
                     lambda n, t: (n, (t * tlc + tlc) % _LC8, 0, 0)),
    ]


def _stage_stats(xin_t, xin4, w1, w2):
    return pl.pallas_call(
        _stats_body,
        grid=(N, NTL),
        in_specs=_halo_specs() + [
            pl.BlockSpec((3 * C, CR), lambda n, t: (0, 0)),
            pl.BlockSpec((C, C), lambda n, t: (0, 0)),
        ],
        out_specs=pl.BlockSpec((8, 128), lambda n, t: (0, 0)),
        out_shape=jax.ShapeDtypeStruct((8, 128), F32),
        compiler_params=pltpu.CompilerParams(
            dimension_semantics=("arbitrary", "arbitrary")),
    )(xin_t, xin4, xin4, w1, w2)


def _embed_body(xin, xp8, xn8, w1, w2, rm, sb1, sb2, xt, yt, c0, c1, c2, c3):
    t = pl.program_id(1)
    o1, o2 = _conv_outs(xin[0], xp8, xn8, t, w1, w2)
    a1 = sb1[0:1, :]
    b1 = sb1[1:2, :]
    a2 = sb2[0:1, :]
    b2 = sb2[1:2, :]
    xm = jnp.maximum(o1 * a1 + b1, 0.0)
    ym = jnp.maximum(o2 * a2 + b2, 0.0)
    xt[0] = xm
    yt[0] = ym
    rot = jnp.dot(xm, rm[...], preferred_element_type=F32)
    lanei = lax.broadcasted_iota(jnp.int32, (TLE, 2 * C), 1)
    outs = (c0, c1, c2, c3)
    for h in range(H):
        rh = rot[:, h * C:(h + 1) * C]
        full = jnp.concatenate([rh, -rh], axis=1)
        m = jnp.max(full, axis=1, keepdims=True)
        cand = jnp.where(full == m, lanei, 2 * C + 1)
        code = jnp.min(cand, axis=1, keepdims=True) + h
        outs[h][0] = code


def _stage_embed(xin_t, xin4, w1, w2, rm, sb1, sb2):
    code_sh = jax.ShapeDtypeStruct((N, L, 1), jnp.int32)
    return pl.pallas_call(
        _embed_body,
        grid=(N, NTL),
        in_specs=_halo_specs() + [
            pl.BlockSpec((3 * C, CR), lambda n, t: (0, 0)),
            pl.BlockSpec((C, C), lambda n, t: (0, 0)),
            pl.BlockSpec((CR, H * C), lambda n, t: (0, 0)),
            pl.BlockSpec((8, CR), lambda n, t: (0, 0)),
            pl.BlockSpec((8, C), lambda n, t: (0, 0)),
        ],
        out_specs=[
            pl.BlockSpec((1, TLE, CR), lambda n, t: (n, t, 0)),
            pl.BlockSpec((1, TLE, C), lambda n, t: (n, t, 0)),
            pl.BlockSpec((1, TLE, 1), lambda n, t: (n, t, 0)),
            pl.BlockSpec((1, TLE, 1), lambda n, t: (n, t, 0)),
            pl.BlockSpec((1, TLE, 1), lambda n, t: (n, t, 0)),
            pl.BlockSpec((1, TLE, 1), lambda n, t: (n, t, 0)),
        ],
        out_shape=[
            jax.ShapeDtypeStruct((N, L, CR), F32),
            jax.ShapeDtypeStruct((N, L, C), F32),
            code_sh, code_sh, code_sh, code_sh,
        ],
    )(xin_t, xin4, xin4, w1, w2, rm, sb1, sb2)


def _rank_body(v4, r4, hh):
    vb = v4[0, 0]
    i0 = lax.broadcasted_iota(jnp.int32, (TB // 8, TB // 8), 0)
    i1 = lax.broadcasted_iota(jnp.int32, (TB // 8, TB // 8), 1)
    lts = _f32(i0 < i1)
    bidx = lax.broadcasted_iota(jnp.int32, (NBP, TB // 8), 0)
    rh = jnp.zeros((NBP, 1), F32)
    rows = []
    for s in range(8):
        vrow = vb[s:s + 1, :]
        oh = _f32(bidx == vrow)
        csum = jnp.dot(oh, lts, preferred_element_type=F32)
        rows.append(jnp.sum(oh * (csum + rh), axis=0, keepdims=True))
        rh = rh + jnp.sum(oh, axis=1, keepdims=True)
    r4[0, 0] = jnp.concatenate(rows, axis=0)
    hh[0, 0] = rh


def _stage_rank(v4):
    return pl.pallas_call(
        _rank_body,
        grid=(N, NBL),
        in_specs=[pl.BlockSpec((1, 1, 8, TB // 8), lambda n, t: (n, t, 0, 0))],
        out_specs=[
            pl.BlockSpec((1, 1, 8, TB // 8), lambda n, t: (n, t, 0, 0)),
            pl.BlockSpec((1, 1, NBP, 1), lambda n, t: (n, t, 0, 0)),
        ],
        out_shape=[
            jax.ShapeDtypeStruct((N, NBL, 8, TB // 8), F32),
            jax.ShapeDtypeStruct((N, NBL, NBP, 1), F32),
        ],
    )(v4)


def _offsets_body(hh, bex, gs, run):
    t = pl.program_id(1)

    @pl.when(t == 0)
    def _():
        run[...] = jnp.zeros((NBP, 1), F32)

    cur = run[...]
    bex[0, 0] = cur
    tot = cur + hh[0, 0]
    run[...] = tot
    row = lax.broadcasted_iota(jnp.int32, (NBP, 1), 0)
    incl = tot
    k = 1
    while k < NBP:
        incl = incl + jnp.where(row >= k, pltpu.roll(incl, k, 0), 0.0)
        k *= 2
    gs[0] = incl - tot


def _stage_offsets(hh):
    return pl.pallas_call(
        _offsets_body,
        grid=(N, NBL),
        in_specs=[pl.BlockSpec((1, 1, NBP, 1), lambda n, t: (n, t, 0, 0))],
        out_specs=[
            pl.BlockSpec((1, 1, NBP, 1), lambda n, t: (n, t, 0, 0)),
            pl.BlockSpec((1, NBP, 1), lambda n, t: (n, 0, 0)),
        ],
        out_shape=[
            jax.ShapeDtypeStruct((N, NBL, NBP, 1), F32),
            jax.ShapeDtypeStruct((N, NBP, 1), F32),
        ],
        scratch_shapes=[pltpu.VMEM((NBP, 1), F32)],
        compiler_params=pltpu.CompilerParams(
            dimension_semantics=("arbitrary", "arbitrary")),
    )(hh)


def _pos_body(v4, r4, off, gs, pg):
    vb = v4[0, 0]
    offc = off[0, 0] + gs[0]
    bidx = lax.broadcasted_iota(jnp.int32, (NBP, TB // 8), 0)
    rows = []
    for s in range(8):
        oh = _f32(bidx == vb[s:s + 1, :])
        rows.append(jnp.sum(oh * offc, axis=0, keepdims=True))
    base = jnp.concatenate(rows, axis=0)
    n = pl.program_id(0)
    pg[0, 0] = (base + r4[0, 0]).astype(jnp.int32) + n * J


def _stage_pos(v4, r4, off, gs):
    return pl.pallas_call(
        _pos_body,
        grid=(N, NBL),
        in_specs=[
            pl.BlockSpec((1, 1, 8, TB // 8), lambda n, t: (n, t, 0, 0)),
            pl.BlockSpec((1, 1, 8, TB // 8), lambda n, t: (n, t, 0, 0)),
            pl.BlockSpec((1, 1, NBP, 1), lambda n, t: (n, t, 0, 0)),
            pl.BlockSpec((1, NBP, 1), lambda n, t: (n, 0, 0)),
        ],
        out_specs=pl.BlockSpec((1, 1, 8, TB // 8), lambda n, t: (n, t, 0, 0)),
        out_shape=jax.ShapeDtypeStruct((N, NBL, 8, TB // 8), jnp.int32),
    )(v4, r4, off, gs)


_PERW = NJ // 32
_CH = 1024


def _sc_invert(pg):
    mesh = plsc.VectorSubcoreMesh(core_axis_name="c", subcore_axis_name="s")

    @functools.partial(
        pl.kernel,
        mesh=mesh,
        out_type=jax.ShapeDtypeStruct((NJ,), jnp.int32),
        scratch_types=[
            pltpu.VMEM((J,), jnp.int32),
            pltpu.VMEM((_CH,), jnp.int32),
        ],
        compiler_params=pltpu.CompilerParams(needs_layout_passes=False),
    )
    def k(pg_h, src_h, inv_v, chunk_v):
        w = lax.axis_index("s") * 2 + lax.axis_index("c")

        @pl.when(w < N)
        def _():
            n = w

            def outer(cc, carry):
                base = cc * _CH
                pltpu.sync_copy(pg_h.at[pl.ds(n * J + base, _CH)], chunk_v)

                def inner(kk, carry2):
                    idx = chunk_v[pl.ds(kk * 16, 16)] - n * J
                    j = base + kk * 16 + lax.broadcasted_iota(
                        jnp.int32, (16,), 0)
                    val = n * L + lax.bitwise_and(j, L - 1)
                    plsc.store_scatter(inv_v, [idx], val)
                    return carry2

                lax.fori_loop(0, _CH // 16, inner, 0)
                return carry

            lax.fori_loop(0, J // _CH, outer, 0)
            pltpu.sync_copy(inv_v, src_h.at[pl.ds(n * J, J)])

    return k(pg)


def _sc_gather_xy(xt_flat, yt_flat, src_idx):
    mesh = plsc.VectorSubcoreMesh(core_axis_name="c", subcore_axis_name="s")

    @functools.partial(
        pl.kernel,
        mesh=mesh,
        out_type=(jax.ShapeDtypeStruct((NJ, CR), F32),
                  jax.ShapeDtypeStruct((NJ, C), F32)),
        scratch_types=[
            pltpu.VMEM((_CH,), jnp.int32),
            pltpu.VMEM((_CH, CR), F32),
            pltpu.VMEM((_CH, C), F32),
            pltpu.SemaphoreType.DMA,
            pltpu.SemaphoreType.DMA,
        ],
        compiler_params=pltpu.CompilerParams(
            needs_layout_passes=False, use_tc_tiling_on_sc=False),
    )
    def k(xt_h, yt_h, si_h, xs_h, ys_h, idx_v, xv, yv, s1, s2):
        w = lax.axis_index("s") * 2 + lax.axis_index("c")
        q0 = w * _PERW

        def body(cc, carry):
            o = q0 + cc * _CH
            pltpu.sync_copy(si_h.at[pl.ds(o, _CH)], idx_v)
            cx = pltpu.async_copy(xt_h.at[idx_v], xv, s1)
            cy = pltpu.async_copy(yt_h.at[idx_v], yv, s2)
            cx.wait()
            cy.wait()
            pltpu.sync_copy(xv, xs_h.at[pl.ds(o, _CH)])
            pltpu.sync_copy(yv, ys_h.at[pl.ds(o, _CH)])
            return carry

        lax.fori_loop(0, _PERW // _CH, body, 0)

    return k(xt_flat, yt_flat, src_idx)


def _sc_gather(att_flat, pg):
    mesh = plsc.VectorSubcoreMesh(core_axis_name="c", subcore_axis_name="s")

    @functools.partial(
        pl.kernel,
        mesh=mesh,
        out_type=jax.ShapeDtypeStruct((NJ, C), F32),
        scratch_types=[
            pltpu.VMEM((_CH,), jnp.int32),
            pltpu.VMEM((_CH, C), F32),
            pltpu.SemaphoreType.DMA,
        ],
        compiler_params=pltpu.CompilerParams(
            needs_layout_passes=False, use_tc_tiling_on_sc=False),
    )
    def k(att_h, pg_h, out_h, idx_v, rows_v, sem):
        w = lax.axis_index("s") * 2 + lax.axis_index("c")
        jg0 = w * _PERW

        def body(cc, carry):
            o = cc * _CH
            pltpu.sync_copy(pg_h.at[pl.ds(jg0 + o, _CH)], idx_v)
            pltpu.async_copy(att_h.at[idx_v], rows_v, sem).wait()
            pltpu.sync_copy(rows_v, out_h.at[pl.ds(jg0 + o, _CH)])
            return carry

        lax.fori_loop(0, _PERW // _CH, body, 0)

    return k(att_flat, pg)


def _att_body(xc, xpl, xnf, yc, ypl, ynf, att, bs):
    xq = xc[0, 0]
    xk = jnp.concatenate([xpl[0, 0], xq, xnf[0, 0]], axis=0)
    yk = jnp.concatenate([ypl[0, 0], yc[0, 0], ynf[0, 0]], axis=0)
    nrm = jnp.sqrt(jnp.sum(xk * xk, axis=1, keepdims=True))
    xk = xk / jnp.maximum(nrm, 5e-5)
    s = lax.dot_general(xq, xk, (((1,), (1,)), ((), ())),
                        preferred_element_type=F32)
    qc = lax.broadcasted_iota(jnp.int32, (RT, RT + 16), 0) // CHUNK
    kc = lax.broadcasted_iota(jnp.int32, (RT, RT + 16), 1) // CHUNK - 1
    valid = jnp.abs(qc - kc) <= 1
    s = jnp.where(valid, s, NEG)
    m = jnp.max(s, axis=1, keepdims=True)
    e = jnp.exp(s - m)
    ssum = jnp.sum(e, axis=1, keepdims=True)
    att[0, 0] = jnp.dot(e, yk, preferred_element_type=F32) * (1.0 / ssum)
    bs[0, 0] = m + jnp.log(ssum)


def _stage_att(xs, ys):
    xs4 = xs.reshape(NH, NT, RT, CR)
    xs8 = xs.reshape(NH, NT * KT, CHUNK, CR)
    ys4 = ys.reshape(NH, NT, RT, C)
    ys8 = ys.reshape(NH, NT * KT, CHUNK, C)
    nchunks = NT * KT
    return pl.pallas_call(
        _att_body,
        grid=(NH, NT),
        in_specs=[
            pl.BlockSpec((1, 1, RT, CR), lambda i, t: (i, t, 0, 0)),
            pl.BlockSpec((1, 1, CHUNK, CR),
                         lambda i, t: (i, (t * KT - 1) % nchunks, 0, 0)),
            pl.BlockSpec((1, 1, CHUNK, CR),
                         lambda i, t: (i, (t * KT + KT) % nchunks, 0, 0)),
            pl.BlockSpec((1, 1, RT, C), lambda i, t: (i, t, 0, 0)),
            pl.BlockSpec((1, 1, CHUNK, C),
                         lambda i, t: (i, (t * KT - 1) % nchunks, 0, 0)),
            pl.BlockSpec((1, 1, CHUNK, C),
                         lambda i, t: (i, (t * KT + KT) % nchunks, 0, 0)),
        ],
        out_specs=[
            pl.BlockSpec((1, 1, RT, C), lambda i, t: (i, t, 0, 0)),
            pl.BlockSpec((1, 1, RT, 1), lambda i, t: (i, t, 0, 0)),
        ],
        out_shape=[
            jax.ShapeDtypeStruct((NH, NT, RT, C), F32),
            jax.ShapeDtypeStruct((NH, NT, RT, 1), F32),
        ],
    )(xs4, xs8, xs8, ys4, ys8, ys8)


_TL = 2048


def _combine_body(attu, bsr, fout, stats):
    b = bsr[0]
    m = jnp.max(b, axis=0, keepdims=True)
    e = jnp.exp(b - m)
    wgt = e / jnp.sum(e, axis=0, keepdims=True)
    fv = jnp.sum(attu[0] * wgt, axis=0)
    fout[0] = fv
    ps = jnp.sum(fv, axis=0, keepdims=True)
    pq = jnp.sum(fv * fv, axis=0, keepdims=True)
    z = jnp.zeros((1, 128 - C), F32)
    blk = jnp.concatenate([
        jnp.concatenate([ps, z], axis=1),
        jnp.concatenate([pq, z], axis=1),
        jnp.zeros((6, 128), F32),
    ], axis=0)
    first = jnp.logical_and(pl.program_id(0) == 0, pl.program_id(1) == 0)

    @pl.when(first)
    def _():
        stats[...] = blk

    @pl.when(jnp.logical_not(first))
    def _():
        stats[...] = stats[...] + blk


def _stage_combine(attu4, bs4):
    return pl.pallas_call(
        _combine_body,
        grid=(N, L // _TL),
        in_specs=[
            pl.BlockSpec((1, H, _TL, C), lambda n, t: (n, 0, t, 0)),
            pl.BlockSpec((1, H, _TL, 1), lambda n, t: (n, 0, t, 0)),
        ],
        out_specs=[
            pl.BlockSpec((1, _TL, C), lambda n, t: (n, t, 0)),
            pl.BlockSpec((8, 128), lambda n, t: (0, 0)),
        ],
        out_shape=[
            jax.ShapeDtypeStruct((N, L, C), F32),
            jax.ShapeDtypeStruct((8, 128), F32),
        ],
        compiler_params=pltpu.CompilerParams(
            dimension_semantics=("arbitrary", "arbitrary")),
    )(attu4, bs4)


def _final_body(fin, xin, sb3, out):
    a3 = sb3[0:1, :]
    b3 = sb3[1:2, :]
    out[0] = fin[0] * a3 + b3 + xin[0]


def _stage_final(fv, xin_t, sb3):
    return pl.pallas_call(
        _final_body,
        grid=(N, L // _TL),
        in_specs=[
            pl.BlockSpec((1, _TL, C), lambda n, t: (n, t, 0)),
            pl.BlockSpec((1, _TL, C), lambda n, t: (n, t, 0)),
            pl.BlockSpec((8, C), lambda n, t: (0, 0)),
        ],
        out_specs=pl.BlockSpec((1, _TL, C), lambda n, t: (n, t, 0)),
        out_shape=jax.ShapeDtypeStruct((N, L, C), F32),
    )(fv, xin_t, sb3)


def _fold_bn(g, b, s, q, cnt, eps=1e-5):
    mean = s / cnt
    var = q / cnt - mean * mean
    a = g / jnp.sqrt(var + eps)
    return a, b - a * mean


def _pack_sb(a, b, width):
    sb = jnp.zeros((8, width), F32)
    return sb.at[0].set(a).at[1].set(b)


def kernel(input_tensor, conv_match_w, bn1_g, bn1_b, conv_asm_w, bn2_g,
           bn2_b, bn3_g, bn3_b, random_rotations):
    xin_t = input_tensor.transpose(0, 2, 1)
    w1 = conv_match_w.transpose(2, 1, 0).reshape(3 * C, CR)
    w2 = conv_asm_w[:, :, 0].transpose(1, 0)
    rm = random_rotations.reshape(CR, H * C)

    xin4 = xin_t.reshape(N, L // 8, 8, C)
    stats = _stage_stats(xin_t, xin4, w1, w2)
    cnt = float(N * L)
    a1, b1 = _fold_bn(bn1_g, bn1_b, stats[0, :CR], stats[1, :CR], cnt)
    a2, b2 = _fold_bn(bn2_g, bn2_b, stats[2, :C], stats[3, :C], cnt)
    xt, yt, c0, c1, c2, c3 = _stage_embed(
        xin_t, xin4, w1, w2, rm, _pack_sb(a1, b1, CR), _pack_sb(a2, b2, C))

    v = jnp.concatenate([c0, c1, c2, c3], axis=2)
    v = v.transpose(0, 2, 1).reshape(N, J)
    v4 = v.reshape(N, NBL, 8, TB // 8)
    r4, hh = _stage_rank(v4)
    off, gs = _stage_offsets(hh)
    pg = _stage_pos(v4, r4, off, gs).reshape(NJ)

    src_idx = _sc_invert(pg)
    xs, ys = _sc_gather_xy(xt.reshape(N * L, CR), yt.reshape(N * L, C),
                           src_idx)
    att4, bs4 = _stage_att(xs, ys)
    attu = _sc_gather(att4.reshape(NJ, C), pg)

    fv, stats3 = _stage_combine(attu.reshape(N, H, L, C),
                                bs4.reshape(N, H, L, 1))
    a3, b3 = _fold_bn(bn3_g, bn3_b, stats3[0, :C], stats3[1, :C], cnt)
    out = _stage_final(fv, xin_t, _pack_sb(a3, b3, C))
    return out.transpose(0, 2, 1)


# speedup vs baseline: 6.0115x; 1.2500x over previous
import functools

import jax
import jax.numpy as jnp
from jax import lax
from jax.experimental import pallas as pl
from jax.experimental.pallas import tpu as pltpu
from jax.experimental.pallas import tpu_sc as plsc

N = 4
C = 64
L = 8192
CR = 16
H = 4
CHUNK = 8
J = H * L
NJ = N * J
NBP = 136
TB = 2048
NBL = J // TB
KT = 32
RT = KT * CHUNK
NT = (L // CHUNK) // KT
NH = N * H
F32 = jnp.float32
NEG = -1e30


def _f32(x):
    return x.astype(F32)


TLE = 2048
NTL = L // TLE
_LC8 = L // 8


def _shifted(xc, xprev8, xnext8, t):
    row = lax.broadcasted_iota(jnp.int32, (TLE, 1), 0)
    prev_row = xprev8[0, 0, 7:8, :] * jnp.where(t == 0, 0.0, 1.0)
    next_row = xnext8[0, 0, 0:1, :] * jnp.where(t == NTL - 1, 0.0, 1.0)
    xm1 = jnp.where(row == 0, prev_row, pltpu.roll(xc, 1, 0))
    xp1 = jnp.where(row == TLE - 1, next_row, pltpu.roll(xc, TLE - 1, 0))
    return xm1, xp1


def _conv_outs(xc, xprev8, xnext8, t, w1, w2):
    xm1, xp1 = _shifted(xc, xprev8, xnext8, t)
    xcat = jnp.concatenate([xm1, xc, xp1], axis=1)
    o1 = jnp.dot(xcat, w1[...], preferred_element_type=F32)
    o2 = jnp.dot(xc, w2[...], preferred_element_type=F32)
    return o1, o2


def _stats_body(xin, xp8, xn8, w1, w2, stats):
    t = pl.program_id(1)
    o1, o2 = _conv_outs(xin[0], xp8, xn8, t, w1, w2)
    s1 = jnp.sum(o1, axis=0, keepdims=True)
    q1 = jnp.sum(o1 * o1, axis=0, keepdims=True)
    s2 = jnp.sum(o2, axis=0, keepdims=True)
    q2 = jnp.sum(o2 * o2, axis=0, keepdims=True)
    z1 = jnp.zeros((1, 128 - CR), F32)
    z2 = jnp.zeros((1, 128 - C), F32)
    blk = jnp.concatenate([
        jnp.concatenate([s1, z1], axis=1),
        jnp.concatenate([q1, z1], axis=1),
        jnp.concatenate([s2, z2], axis=1),
        jnp.concatenate([q2, z2], axis=1),
        jnp.zeros((4, 128), F32),
    ], axis=0)
    first = jnp.logical_and(pl.program_id(0) == 0, t == 0)

    @pl.when(first)
    def _():
        stats[...] = blk

    @pl.when(jnp.logical_not(first))
    def _():
        stats[...] = stats[...] + blk


def _halo_specs():
    tlc = TLE // 8
    return [
        pl.BlockSpec((1, TLE, C), lambda n, t: (n, t, 0)),
        pl.BlockSpec((1, 1, 8, C),
                     lambda n, t: (n, (t * tlc - 1) % _LC8, 0, 0)),
        pl.BlockSpec((1, 1, 8, C),
                     lambda n, t: (n, (t * tlc + tlc) % _LC8, 0, 0)),
    ]


def _stage_stats(xin_t, xin4, w1, w2):
    return pl.pallas_call(
        _stats_body,
        grid=(N, NTL),
        in_specs=_halo_specs() + [
            pl.BlockSpec((3 * C, CR), lambda n, t: (0, 0)),
            pl.BlockSpec((C, C), lambda n, t: (0, 0)),
        ],
        out_specs=pl.BlockSpec((8, 128), lambda n, t: (0, 0)),
        out_shape=jax.ShapeDtypeStruct((8, 128), F32),
        compiler_params=pltpu.CompilerParams(
            dimension_semantics=("arbitrary", "arbitrary")),
    )(xin_t, xin4, xin4, w1, w2)


def _embed_body(xin, xp8, xn8, w1, w2, rm, sb1, sb2, xy, c0, c1, c2, c3):
    t = pl.program_id(1)
    o1, o2 = _conv_outs(xin[0], xp8, xn8, t, w1, w2)
    a1 = sb1[0:1, :]
    b1 = sb1[1:2, :]
    a2 = sb2[0:1, :]
    b2 = sb2[1:2, :]
    xm = jnp.maximum(o1 * a1 + b1, 0.0)
    ym = jnp.maximum(o2 * a2 + b2, 0.0)
    xy[0] = jnp.concatenate([ym, xm, jnp.zeros((TLE, 128 - C - CR), F32)],
                            axis=1)
    rot = jnp.dot(xm, rm[...], preferred_element_type=F32)
    lanei = lax.broadcasted_iota(jnp.int32, (TLE, 2 * C), 1)
    outs = (c0, c1, c2, c3)
    for h in range(H):
        rh = rot[:, h * C:(h + 1) * C]
        full = jnp.concatenate([rh, -rh], axis=1)
        m = jnp.max(full, axis=1, keepdims=True)
        cand = jnp.where(full == m, lanei, 2 * C + 1)
        code = jnp.min(cand, axis=1, keepdims=True) + h
        outs[h][0] = code


def _stage_embed(xin_t, xin4, w1, w2, rm, sb1, sb2):
    code_sh = jax.ShapeDtypeStruct((N, L, 1), jnp.int32)
    return pl.pallas_call(
        _embed_body,
        grid=(N, NTL),
        in_specs=_halo_specs() + [
            pl.BlockSpec((3 * C, CR), lambda n, t: (0, 0)),
            pl.BlockSpec((C, C), lambda n, t: (0, 0)),
            pl.BlockSpec((CR, H * C), lambda n, t: (0, 0)),
            pl.BlockSpec((8, CR), lambda n, t: (0, 0)),
            pl.BlockSpec((8, C), lambda n, t: (0, 0)),
        ],
        out_specs=[
            pl.BlockSpec((1, TLE, 128), lambda n, t: (n, t, 0)),
            pl.BlockSpec((1, TLE, 1), lambda n, t: (n, t, 0)),
            pl.BlockSpec((1, TLE, 1), lambda n, t: (n, t, 0)),
            pl.BlockSpec((1, TLE, 1), lambda n, t: (n, t, 0)),
            pl.BlockSpec((1, TLE, 1), lambda n, t: (n, t, 0)),
        ],
        out_shape=[
            jax.ShapeDtypeStruct((N, L, 128), F32),
            code_sh, code_sh, code_sh, code_sh,
        ],
    )(xin_t, xin4, xin4, w1, w2, rm, sb1, sb2)


def _rank_body(v4, r4, hh):
    vb = v4[0, 0]
    i0 = lax.broadcasted_iota(jnp.int32, (TB // 8, TB // 8), 0)
    i1 = lax.broadcasted_iota(jnp.int32, (TB // 8, TB // 8), 1)
    lts = _f32(i0 < i1)
    bidx = lax.broadcasted_iota(jnp.int32, (NBP, TB // 8), 0)
    rh = jnp.zeros((NBP, 1), F32)
    rows = []
    for s in range(8):
        vrow = vb[s:s + 1, :]
        oh = _f32(bidx == vrow)
        csum = jnp.dot(oh, lts, preferred_element_type=F32)
        rows.append(jnp.sum(oh * (csum + rh), axis=0, keepdims=True))
        rh = rh + jnp.sum(oh, axis=1, keepdims=True)
    r4[0, 0] = jnp.concatenate(rows, axis=0)
    hh[0, 0] = rh


def _stage_rank(v4):
    return pl.pallas_call(
        _rank_body,
        grid=(N, NBL),
        in_specs=[pl.BlockSpec((1, 1, 8, TB // 8), lambda n, t: (n, t, 0, 0))],
        out_specs=[
            pl.BlockSpec((1, 1, 8, TB // 8), lambda n, t: (n, t, 0, 0)),
            pl.BlockSpec((1, 1, NBP, 1), lambda n, t: (n, t, 0, 0)),
        ],
        out_shape=[
            jax.ShapeDtypeStruct((N, NBL, 8, TB // 8), F32),
            jax.ShapeDtypeStruct((N, NBL, NBP, 1), F32),
        ],
    )(v4)


def _offsets_body(hh, bex, gs, run):
    t = pl.program_id(1)

    @pl.when(t == 0)
    def _():
        run[...] = jnp.zeros((NBP, 1), F32)

    cur = run[...]
    bex[0, 0] = cur
    tot = cur + hh[0, 0]
    run[...] = tot
    row = lax.broadcasted_iota(jnp.int32, (NBP, 1), 0)
    incl = tot
    k = 1
    while k < NBP:
        incl = incl + jnp.where(row >= k, pltpu.roll(incl, k, 0), 0.0)
        k *= 2
    gs[0] = incl - tot


def _stage_offsets(hh):
    return pl.pallas_call(
        _offsets_body,
        grid=(N, NBL),
        in_specs=[pl.BlockSpec((1, 1, NBP, 1), lambda n, t: (n, t, 0, 0))],
        out_specs=[
            pl.BlockSpec((1, 1, NBP, 1), lambda n, t: (n, t, 0, 0)),
            pl.BlockSpec((1, NBP, 1), lambda n, t: (n, 0, 0)),
        ],
        out_shape=[
            jax.ShapeDtypeStruct((N, NBL, NBP, 1), F32),
            jax.ShapeDtypeStruct((N, NBP, 1), F32),
        ],
        scratch_shapes=[pltpu.VMEM((NBP, 1), F32)],
        compiler_params=pltpu.CompilerParams(
            dimension_semantics=("arbitrary", "arbitrary")),
    )(hh)


def _pos_body(v4, r4, off, gs, pg):
    vb = v4[0, 0]
    offc = off[0, 0] + gs[0]
    bidx = lax.broadcasted_iota(jnp.int32, (NBP, TB // 8), 0)
    rows = []
    for s in range(8):
        oh = _f32(bidx == vb[s:s + 1, :])
        rows.append(jnp.sum(oh * offc, axis=0, keepdims=True))
    base = jnp.concatenate(rows, axis=0)
    n = pl.program_id(0)
    pg[0, 0] = (base + r4[0, 0]).astype(jnp.int32) + n * J


def _stage_pos(v4, r4, off, gs):
    return pl.pallas_call(
        _pos_body,
        grid=(N, NBL),
        in_specs=[
            pl.BlockSpec((1, 1, 8, TB // 8), lambda n, t: (n, t, 0, 0)),
            pl.BlockSpec((1, 1, 8, TB // 8), lambda n, t: (n, t, 0, 0)),
            pl.BlockSpec((1, 1, NBP, 1), lambda n, t: (n, t, 0, 0)),
            pl.BlockSpec((1, NBP, 1), lambda n, t: (n, 0, 0)),
        ],
        out_specs=pl.BlockSpec((1, 1, 8, TB // 8), lambda n, t: (n, t, 0, 0)),
        out_shape=jax.ShapeDtypeStruct((N, NBL, 8, TB // 8), jnp.int32),
    )(v4, r4, off, gs)


_PERW = NJ // 32
_CH = 1024


def _sc_invert(pg):
    mesh = plsc.VectorSubcoreMesh(core_axis_name="c", subcore_axis_name="s")

    @functools.partial(
        pl.kernel,
        mesh=mesh,
        out_type=jax.ShapeDtypeStruct((NJ,), jnp.int32),
        scratch_types=[
            pltpu.VMEM((J,), jnp.int32),
            pltpu.VMEM((_CH,), jnp.int32),
        ],
        compiler_params=pltpu.CompilerParams(needs_layout_passes=False),
    )
    def k(pg_h, src_h, inv_v, chunk_v):
        w = lax.axis_index("s") * 2 + lax.axis_index("c")

        @pl.when(w < N)
        def _():
            n = w

            def outer(cc, carry):
                base = cc * _CH
                pltpu.sync_copy(pg_h.at[pl.ds(n * J + base, _CH)], chunk_v)

                def inner(kk, carry2):
                    idx = chunk_v[pl.ds(kk * 16, 16)] - n * J
                    j = base + kk * 16 + lax.broadcasted_iota(
                        jnp.int32, (16,), 0)
                    val = n * L + lax.bitwise_and(j, L - 1)
                    plsc.store_scatter(inv_v, [idx], val)
                    return carry2

                lax.fori_loop(0, _CH // 16, inner, 0)
                return carry

            lax.fori_loop(0, J // _CH, outer, 0)
            pltpu.sync_copy(inv_v, src_h.at[pl.ds(n * J, J)])

    return k(pg)


_CHG = 512


def _sc_gather_rows(table, idx):
    nr = idx.shape[0]
    perw = nr // 32
    mesh = plsc.VectorSubcoreMesh(core_axis_name="c", subcore_axis_name="s")

    @functools.partial(
        pl.kernel,
        mesh=mesh,
        out_type=jax.ShapeDtypeStruct((nr, 128), F32),
        scratch_types=[
            pltpu.VMEM((_CHG,), jnp.int32),
            pltpu.VMEM((_CHG, 128), F32),
            pltpu.SemaphoreType.DMA,
        ],
    )
    def k(tbl_h, idx_h, out_h, idx_v, rows_v, sem):
        w = lax.axis_index("s") * 2 + lax.axis_index("c")
        q0 = w * perw

        def body(cc, carry):
            o = q0 + cc * _CHG
            pltpu.sync_copy(idx_h.at[pl.ds(o, _CHG)], idx_v)
            pltpu.async_copy(tbl_h.at[idx_v], rows_v, sem).wait()
            pltpu.sync_copy(rows_v, out_h.at[pl.ds(o, _CHG)])
            return carry

        lax.fori_loop(0, perw // _CHG, body, 0)

    return k(table, idx)


def _att_body(xyc, xypl, xynf, att, bs):
    cur = xyc[0, 0]
    prv = xypl[0, 0]
    nxt = xynf[0, 0]
    xq = cur[:, C:C + CR]
    xk = jnp.concatenate([prv[:, C:C + CR], xq, nxt[:, C:C + CR]], axis=0)
    yk = jnp.concatenate([prv[:, :C], cur[:, :C], nxt[:, :C]], axis=0)
    nrm = jnp.sqrt(jnp.sum(xk * xk, axis=1, keepdims=True))
    xk = xk / jnp.maximum(nrm, 5e-5)
    s = lax.dot_general(xq, xk, (((1,), (1,)), ((), ())),
                        preferred_element_type=F32)
    qc = lax.broadcasted_iota(jnp.int32, (RT, RT + 16), 0) // CHUNK
    kc = lax.broadcasted_iota(jnp.int32, (RT, RT + 16), 1) // CHUNK - 1
    valid = jnp.abs(qc - kc) <= 1
    s = jnp.where(valid, s, NEG)
    m = jnp.max(s, axis=1, keepdims=True)
    e = jnp.exp(s - m)
    ssum = jnp.sum(e, axis=1, keepdims=True)
    av = jnp.dot(e, yk, preferred_element_type=F32) * (1.0 / ssum)
    att[0, 0] = jnp.concatenate([av, jnp.zeros((RT, 128 - C), F32)], axis=1)
    bs[0, 0] = m + jnp.log(ssum)


def _stage_att(xys):
    xy4 = xys.reshape(NH, NT, RT, 128)
    xy8 = xys.reshape(NH, NT * KT, CHUNK, 128)
    nchunks = NT * KT
    return pl.pallas_call(
        _att_body,
        grid=(NH, NT),
        in_specs=[
            pl.BlockSpec((1, 1, RT, 128), lambda i, t: (i, t, 0, 0)),
            pl.BlockSpec((1, 1, CHUNK, 128),
                         lambda i, t: (i, (t * KT - 1) % nchunks, 0, 0)),
            pl.BlockSpec((1, 1, CHUNK, 128),
                         lambda i, t: (i, (t * KT + KT) % nchunks, 0, 0)),
        ],
        out_specs=[
            pl.BlockSpec((1, 1, RT, 128), lambda i, t: (i, t, 0, 0)),
            pl.BlockSpec((1, 1, RT, 1), lambda i, t: (i, t, 0, 0)),
        ],
        out_shape=[
            jax.ShapeDtypeStruct((NH, NT, RT, 128), F32),
            jax.ShapeDtypeStruct((NH, NT, RT, 1), F32),
        ],
    )(xy4, xy8, xy8)


_TL = 2048


def _combine_body(attu, bsr, fout, stats):
    b = bsr[0]
    m = jnp.max(b, axis=0, keepdims=True)
    e = jnp.exp(b - m)
    wgt = e / jnp.sum(e, axis=0, keepdims=True)
    fv = jnp.sum(attu[0][:, :, :C] * wgt, axis=0)
    fout[0] = fv
    ps = jnp.sum(fv, axis=0, keepdims=True)
    pq = jnp.sum(fv * fv, axis=0, keepdims=True)
    z = jnp.zeros((1, 128 - C), F32)
    blk = jnp.concatenate([
        jnp.concatenate([ps, z], axis=1),
        jnp.concatenate([pq, z], axis=1),
        jnp.zeros((6, 128), F32),
    ], axis=0)
    first = jnp.logical_and(pl.program_id(0) == 0, pl.program_id(1) == 0)

    @pl.when(first)
    def _():
        stats[...] = blk

    @pl.when(jnp.logical_not(first))
    def _():
        stats[...] = stats[...] + blk


def _stage_combine(attu4, bs4):
    return pl.pallas_call(
        _combine_body,
        grid=(N, L // _TL),
        in_specs=[
            pl.BlockSpec((1, H, _TL, 128), lambda n, t: (n, 0, t, 0)),
            pl.BlockSpec((1, H, _TL, 1), lambda n, t: (n, 0, t, 0)),
        ],
        out_specs=[
            pl.BlockSpec((1, _TL, C), lambda n, t: (n, t, 0)),
            pl.BlockSpec((8, 128), lambda n, t: (0, 0)),
        ],
        out_shape=[
            jax.ShapeDtypeStruct((N, L, C), F32),
            jax.ShapeDtypeStruct((8, 128), F32),
        ],
        compiler_params=pltpu.CompilerParams(
            dimension_semantics=("arbitrary", "arbitrary")),
    )(attu4, bs4)


def _final_body(fin, xin, sb3, out):
    a3 = sb3[0:1, :]
    b3 = sb3[1:2, :]
    out[0] = fin[0] * a3 + b3 + xin[0]


def _stage_final(fv, xin_t, sb3):
    return pl.pallas_call(
        _final_body,
        grid=(N, L // _TL),
        in_specs=[
            pl.BlockSpec((1, _TL, C), lambda n, t: (n, t, 0)),
            pl.BlockSpec((1, _TL, C), lambda n, t: (n, t, 0)),
            pl.BlockSpec((8, C), lambda n, t: (0, 0)),
        ],
        out_specs=pl.BlockSpec((1, _TL, C), lambda n, t: (n, t, 0)),
        out_shape=jax.ShapeDtypeStruct((N, L, C), F32),
    )(fv, xin_t, sb3)


def _fold_bn(g, b, s, q, cnt, eps=1e-5):
    mean = s / cnt
    var = q / cnt - mean * mean
    a = g / jnp.sqrt(var + eps)
    return a, b - a * mean


def _pack_sb(a, b, width):
    sb = jnp.zeros((8, width), F32)
    return sb.at[0].set(a).at[1].set(b)


def kernel(input_tensor, conv_match_w, bn1_g, bn1_b, conv_asm_w, bn2_g,
           bn2_b, bn3_g, bn3_b, random_rotations):
    xin_t = input_tensor.transpose(0, 2, 1)
    w1 = conv_match_w.transpose(2, 1, 0).reshape(3 * C, CR)
    w2 = conv_asm_w[:, :, 0].transpose(1, 0)
    rm = random_rotations.reshape(CR, H * C)

    xin4 = xin_t.reshape(N, L // 8, 8, C)
    stats = _stage_stats(xin_t, xin4, w1, w2)
    cnt = float(N * L)
    a1, b1 = _fold_bn(bn1_g, bn1_b, stats[0, :CR], stats[1, :CR], cnt)
    a2, b2 = _fold_bn(bn2_g, bn2_b, stats[2, :C], stats[3, :C], cnt)
    xy, c0, c1, c2, c3 = _stage_embed(
        xin_t, xin4, w1, w2, rm, _pack_sb(a1, b1, CR), _pack_sb(a2, b2, C))

    v = jnp.concatenate([c0, c1, c2, c3], axis=2)
    v = v.transpose(0, 2, 1).reshape(N, J)
    v4 = v.reshape(N, NBL, 8, TB // 8)
    r4, hh = _stage_rank(v4)
    off, gs = _stage_offsets(hh)
    pg = _stage_pos(v4, r4, off, gs).reshape(NJ)

    src_idx = _sc_invert(pg)
    xys = _sc_gather_rows(xy.reshape(N * L, 128), src_idx)
    att4, bs4 = _stage_att(xys)
    attu = _sc_gather_rows(att4.reshape(NJ, 128), pg)

    fv, stats3 = _stage_combine(attu.reshape(N, H, L, 128),
                                bs4.reshape(N, H, L, 1))
    a3, b3 = _fold_bn(bn3_g, bn3_b, stats3[0, :C], stats3[1, :C], cnt)
    out = _stage_final(fv, xin_t, _pack_sb(a3, b3, C))
    return out.transpose(0, 2, 1)
